# Initial kernel scaffold; baseline (speedup 1.0000x reference)
import jax
import jax.numpy as jnp
from jax.experimental import pallas as pl


def kernel(z_e, codebook_tensor_pca, W, b):
    raise NotImplementedError("write your pallas kernel here")


# speedup vs baseline: 2.8750x; 2.8750x over previous
import jax
import jax.numpy as jnp
from jax.experimental import pallas as pl
from jax.experimental.pallas import tpu as pltpu
from jax.experimental.pallas import tpu_sc as plsc

WORD = 10
D = 512
P = 4096
KCB = 8192
NTOT = 5120
BITEMS = 512
TOPK = 10
PADK = 16


_RK = 512


def _mc_body(c_ref, w_ref, b_ref, mc_ref, cn_ref):
    mc = jax.lax.dot_general(
        c_ref[...], w_ref[...], (((1,), (1,)), ((), ())),
        preferred_element_type=jnp.float32)
    mc = mc + b_ref[...]
    mc_ref[...] = mc
    cn_ref[...] = jnp.sum(mc * mc, axis=1, keepdims=True)


def _map_codebook(cpca, w, b):
    return pl.pallas_call(
        _mc_body,
        grid=(KCB // _RK,),
        in_specs=[
            pl.BlockSpec((_RK, P), lambda i: (i, 0)),
            pl.BlockSpec((D, P), lambda i: (0, 0)),
            pl.BlockSpec((1, D), lambda i: (0, 0)),
        ],
        out_specs=[
            pl.BlockSpec((_RK, D), lambda i: (i, 0)),
            pl.BlockSpec((_RK, 1), lambda i: (i, 0)),
        ],
        out_shape=[
            jax.ShapeDtypeStruct((KCB, D), jnp.float32),
            jax.ShapeDtypeStruct((KCB, 1), jnp.float32),
        ],
    )(cpca, w, b.reshape(1, D))


_BN = 512
_CH = 1024
_NCH = KCB // _CH


def _topk_body(z_ref, mc_ref, cn_ref, vals_ref, idx_ref, scr_ref):
    k = pl.program_id(1)
    dot = jax.lax.dot_general(
        z_ref[...], mc_ref[...], (((1,), (1,)), ((), ())),
        preferred_element_type=jnp.float32)
    zn = jnp.sum(z_ref[...] * z_ref[...], axis=1, keepdims=True)
    scr_ref[k] = (zn + cn_ref[...]) - 2.0 * dot

    @pl.when(k == _NCH - 1)
    def _extract():
        inf = jnp.float32(jnp.inf)
        bigi = jnp.int32(2**30)

        def t_body(t, carry):
            vals_acc, idx_acc, prev_idx = carry

            def min_body(c, m):
                dch = scr_ref[c]
                gio = jax.lax.broadcasted_iota(jnp.int32, (_BN, _CH), 1) + c * _CH
                dch = jnp.where(gio == prev_idx, inf, dch)
                scr_ref[c] = dch
                return jnp.minimum(m, jnp.min(dch, axis=1, keepdims=True))

            m = jax.lax.fori_loop(0, _NCH, min_body,
                                  jnp.full((_BN, 1), inf, jnp.float32))

            def arg_body(c, best):
                dch = scr_ref[c]
                gio = jax.lax.broadcasted_iota(jnp.int32, (_BN, _CH), 1) + c * _CH
                cand = jnp.min(jnp.where(dch == m, gio, bigi), axis=1,
                               keepdims=True)
                return jnp.minimum(best, cand)

            widx = jax.lax.fori_loop(0, _NCH, arg_body,
                                     jnp.full((_BN, 1), bigi, jnp.int32))

            lane = jax.lax.broadcasted_iota(jnp.int32, (_BN, PADK), 1)
            vals_acc = jnp.where(lane == t, m, vals_acc)
            idx_acc = jnp.where(lane == t, widx, idx_acc)
            return vals_acc, idx_acc, widx

        vals_acc, idx_acc, _ = jax.lax.fori_loop(
            0, TOPK, t_body,
            (jnp.full((_BN, PADK), inf, jnp.float32),
             jnp.zeros((_BN, PADK), jnp.int32),
             jnp.full((_BN, 1), -1, jnp.int32)))
        vals_ref[...] = vals_acc
        idx_ref[...] = idx_acc


def _topk(z_e, mc, cn_row):
    return pl.pallas_call(
        _topk_body,
        grid=(NTOT // _BN, _NCH),
        in_specs=[
            pl.BlockSpec((_BN, D), lambda i, k: (i, 0)),
            pl.BlockSpec((_CH, D), lambda i, k: (k, 0)),
            pl.BlockSpec((1, _CH), lambda i, k: (0, k)),
        ],
        out_specs=[
            pl.BlockSpec((_BN, PADK), lambda i, k: (i, 0)),
            pl.BlockSpec((_BN, PADK), lambda i, k: (i, 0)),
        ],
        out_shape=[
            jax.ShapeDtypeStruct((NTOT, PADK), jnp.float32),
            jax.ShapeDtypeStruct((NTOT, PADK), jnp.int32),
        ],
        scratch_shapes=[pltpu.VMEM((_NCH, _BN, _CH), jnp.float32)],
    )(z_e, mc, cn_row)


def _select_body(vals_ref, idx_ref, out_ref):
    inf = jnp.float32(jnp.inf)
    bigi = jnp.int32(2**30)
    lane = jax.lax.broadcasted_iota(jnp.int32, (BITEMS, PADK), 1)
    acc = jnp.zeros((BITEMS, PADK), jnp.int32)
    chosen = []
    for i in range(WORD):
        cv = vals_ref[:, PADK * i:PADK * (i + 1)]
        ci = idx_ref[:, PADK * i:PADK * (i + 1)]
        banned = jnp.zeros(cv.shape, jnp.bool_)
        for cj in chosen:
            banned = banned | (ci == cj)
        mv = jnp.where(banned, inf, cv)
        m = jnp.min(mv, axis=1, keepdims=True)
        pick = jnp.min(jnp.where(mv == m, ci, bigi), axis=1, keepdims=True)
        chosen.append(pick)
        acc = jnp.where(lane == i, pick, acc)
    out_ref[...] = acc


def _select(vals, idxs):
    return pl.pallas_call(
        _select_body,
        grid=(1,),
        in_specs=[
            pl.BlockSpec((BITEMS, WORD * PADK), lambda i: (0, 0)),
            pl.BlockSpec((BITEMS, WORD * PADK), lambda i: (0, 0)),
        ],
        out_specs=pl.BlockSpec((BITEMS, PADK), lambda i: (0, 0)),
        out_shape=jax.ShapeDtypeStruct((BITEMS, PADK), jnp.int32),
    )(vals.reshape(BITEMS, WORD * PADK), idxs.reshape(BITEMS, WORD * PADK))


_GW = 128
_DH = D // 2


def _gather_rows(mc_half, idx2):
    @pl.kernel(
        out_type=jax.ShapeDtypeStruct((NTOT, _DH), jnp.float32),
        mesh=plsc.VectorSubcoreMesh(core_axis_name="core",
                                    subcore_axis_name="subcore"),
    )
    def _k(mc_hbm, i_hbm, o_hbm):
        def body(i_vmem, o_vmem):
            pltpu.sync_copy(mc_hbm.at[i_vmem.at[0]], o_vmem)

        pltpu.emit_pipeline(
            body,
            grid=(NTOT // _GW,),
            in_specs=[pl.BlockSpec((1, _GW), index_map=lambda i: (0, i))],
            out_specs=[pl.BlockSpec((_GW, _DH), index_map=lambda i: (i, 0))],
            core_axis_name="subcore",
            dimension_semantics=(pltpu.PARALLEL,),
        )(i_hbm, o_hbm)

    return _k(mc_half, idx2)


_BE = 512


def _st_body(zqa_ref, zqb_ref, ze_ref, out_ref, loss_ref):
    i = pl.program_id(0)
    ze = ze_ref[...]
    zq = jnp.concatenate([zqa_ref[...], zqb_ref[...]], axis=1)
    diff = zq - ze
    out_ref[...] = ze + diff
    part = jnp.sum(diff * diff).reshape(1, 1)

    @pl.when(i == 0)
    def _init():
        loss_ref[...] = part

    @pl.when(i != 0)
    def _acc():
        loss_ref[...] = loss_ref[...] + part


def _st_loss(z_qa, z_qb, z_e):
    return pl.pallas_call(
        _st_body,
        grid=(NTOT // _BE,),
        in_specs=[
            pl.BlockSpec((_BE, _DH), lambda i: (i, 0)),
            pl.BlockSpec((_BE, _DH), lambda i: (i, 0)),
            pl.BlockSpec((_BE, D), lambda i: (i, 0)),
        ],
        out_specs=[
            pl.BlockSpec((_BE, D), lambda i: (i, 0)),
            pl.BlockSpec((1, 1), lambda i: (0, 0)),
        ],
        out_shape=[
            jax.ShapeDtypeStruct((NTOT, D), jnp.float32),
            jax.ShapeDtypeStruct((1, 1), jnp.float32),
        ],
    )(z_qa, z_qb, z_e)


def kernel(z_e, codebook_tensor_pca, W, b):
    mc, cn = _map_codebook(codebook_tensor_pca, W, b)
    vals, idxs = _topk(z_e, mc, cn.reshape(1, KCB))
    sel = _select(vals, idxs)
    idx2 = sel[:, :WORD].reshape(1, NTOT)
    z_qa = _gather_rows(mc[:, :_DH], idx2)
    z_qb = _gather_rows(mc[:, _DH:], idx2)
    z_q_st, loss_sum = _st_loss(z_qa, z_qb, z_e)
    m = loss_sum[0, 0] / jnp.float32(NTOT * D)
    loss = 0.75 * m + 0.25 * m
    return (z_q_st, loss)


# speedup vs baseline: 3.9317x; 1.3676x over previous
import jax
import jax.numpy as jnp
from jax.experimental import pallas as pl
from jax.experimental.pallas import tpu as pltpu
from jax.experimental.pallas import tpu_sc as plsc

WORD = 10
D = 512
P = 4096
KCB = 8192
NTOT = 5120
BITEMS = 512
TOPK = 10
PADK = 16


_RK = 512


def _mc_body(c_ref, w_ref, b_ref, mc_ref, cn_ref):
    mc = jax.lax.dot_general(
        c_ref[...], w_ref[...], (((1,), (1,)), ((), ())),
        preferred_element_type=jnp.float32)
    mc = mc + b_ref[...]
    mc_ref[...] = mc
    cn_ref[...] = jnp.sum(mc * mc, axis=1, keepdims=True)


def _map_codebook(cpca, w, b):
    return pl.pallas_call(
        _mc_body,
        grid=(KCB // _RK,),
        in_specs=[
            pl.BlockSpec((_RK, P), lambda i: (i, 0)),
            pl.BlockSpec((D, P), lambda i: (0, 0)),
            pl.BlockSpec((1, D), lambda i: (0, 0)),
        ],
        out_specs=[
            pl.BlockSpec((_RK, D), lambda i: (i, 0)),
            pl.BlockSpec((_RK, 1), lambda i: (i, 0)),
        ],
        out_shape=[
            jax.ShapeDtypeStruct((KCB, D), jnp.float32),
            jax.ShapeDtypeStruct((KCB, 1), jnp.float32),
        ],
    )(cpca, w, b.reshape(1, D))


_BN = 512
_CH = 1024
_NCH = KCB // _CH


def _topk_body(z_ref, mc_ref, cn_ref, vals_ref, idx_ref, scr_ref):
    s = pl.program_id(0)
    k = pl.program_id(1)
    dot = jax.lax.dot_general(
        z_ref[...], mc_ref[...], (((1,), (1,)), ((), ())),
        preferred_element_type=jnp.float32)
    zn = jnp.sum(z_ref[...] * z_ref[...], axis=1, keepdims=True)
    scr_ref[k] = (zn + cn_ref[...]) - 2.0 * dot

    @pl.when(k == _NCH - 1)
    def _extract():
        inf = jnp.float32(jnp.inf)
        bigi = jnp.int32(2**30)
        tmax = s + 1

        def t_body(t, carry):
            vals_acc, idx_acc, prev_idx = carry

            def chunk_body(c, mc_carry):
                m, a = mc_carry
                dch = scr_ref[c]
                gio = (jax.lax.broadcasted_iota(jnp.int32, (_BN, _CH), 1)
                       + c * _CH)
                dch = jnp.where(gio == prev_idx, inf, dch)

                @pl.when(jnp.logical_and(t > 0, t < tmax - 1))
                def _persist():
                    scr_ref[c] = dch

                cm = jnp.min(dch, axis=1, keepdims=True)
                ca = jnp.min(jnp.where(dch == cm, gio, bigi), axis=1,
                             keepdims=True)
                take = cm < m
                return jnp.where(take, cm, m), jnp.where(take, ca, a)

            m, widx = jax.lax.fori_loop(
                0, _NCH, chunk_body,
                (jnp.full((_BN, 1), inf, jnp.float32),
                 jnp.full((_BN, 1), bigi, jnp.int32)))

            lane = jax.lax.broadcasted_iota(jnp.int32, (_BN, PADK), 1)
            vals_acc = jnp.where(lane == t, m, vals_acc)
            idx_acc = jnp.where(lane == t, widx, idx_acc)
            return vals_acc, idx_acc, widx

        vals_acc, idx_acc, _ = jax.lax.fori_loop(
            0, tmax, t_body,
            (jnp.full((_BN, PADK), inf, jnp.float32),
             jnp.zeros((_BN, PADK), jnp.int32),
             jnp.full((_BN, 1), -1, jnp.int32)))
        vals_ref[...] = vals_acc
        idx_ref[...] = idx_acc


def _topk(z_slot, mc, cn_row):
    return pl.pallas_call(
        _topk_body,
        grid=(NTOT // _BN, _NCH),
        in_specs=[
            pl.BlockSpec((_BN, D), lambda i, k: (i, 0)),
            pl.BlockSpec((_CH, D), lambda i, k: (k, 0)),
            pl.BlockSpec((1, _CH), lambda i, k: (0, k)),
        ],
        out_specs=[
            pl.BlockSpec((_BN, PADK), lambda i, k: (i, 0)),
            pl.BlockSpec((_BN, PADK), lambda i, k: (i, 0)),
        ],
        out_shape=[
            jax.ShapeDtypeStruct((NTOT, PADK), jnp.float32),
            jax.ShapeDtypeStruct((NTOT, PADK), jnp.int32),
        ],
        scratch_shapes=[pltpu.VMEM((_NCH, _BN, _CH), jnp.float32)],
    )(z_slot, mc, cn_row)


def _select_body(vals_ref, idx_ref, out_ref):
    inf = jnp.float32(jnp.inf)
    bigi = jnp.int32(2**30)
    lane = jax.lax.broadcasted_iota(jnp.int32, (BITEMS, PADK), 1)
    acc = jnp.zeros((BITEMS, PADK), jnp.int32)
    chosen = []
    for i in range(WORD):
        cv = vals_ref[:, PADK * i:PADK * (i + 1)]
        ci = idx_ref[:, PADK * i:PADK * (i + 1)]
        banned = jnp.zeros(cv.shape, jnp.bool_)
        for cj in chosen:
            banned = banned | (ci == cj)
        mv = jnp.where(banned, inf, cv)
        m = jnp.min(mv, axis=1, keepdims=True)
        pick = jnp.min(jnp.where(mv == m, ci, bigi), axis=1, keepdims=True)
        chosen.append(pick)
        acc = jnp.where(lane == i, pick, acc)
    out_ref[...] = acc


def _select(vals, idxs):
    return pl.pallas_call(
        _select_body,
        grid=(1,),
        in_specs=[
            pl.BlockSpec((BITEMS, WORD * PADK), lambda i: (0, 0)),
            pl.BlockSpec((BITEMS, WORD * PADK), lambda i: (0, 0)),
        ],
        out_specs=pl.BlockSpec((BITEMS, PADK), lambda i: (0, 0)),
        out_shape=jax.ShapeDtypeStruct((BITEMS, PADK), jnp.int32),
    )(vals.reshape(BITEMS, WORD * PADK), idxs.reshape(BITEMS, WORD * PADK))


_GW = 128
_DH = D // 2


def _gather_rows(mc_half, idx2):
    @pl.kernel(
        out_type=jax.ShapeDtypeStruct((NTOT, _DH), jnp.float32),
        mesh=plsc.VectorSubcoreMesh(core_axis_name="core",
                                    subcore_axis_name="subcore"),
    )
    def _k(mc_hbm, i_hbm, o_hbm):
        def body(i_vmem, o_vmem):
            pltpu.sync_copy(mc_hbm.at[i_vmem.at[0]], o_vmem)

        pltpu.emit_pipeline(
            body,
            grid=(NTOT // _GW,),
            in_specs=[pl.BlockSpec((1, _GW), index_map=lambda i: (0, i))],
            out_specs=[pl.BlockSpec((_GW, _DH), index_map=lambda i: (i, 0))],
            core_axis_name="subcore",
            dimension_semantics=(pltpu.PARALLEL,),
        )(i_hbm, o_hbm)

    return _k(mc_half, idx2)


_BE = 512


def _st_body(zqa_ref, zqb_ref, ze_ref, out_ref, loss_ref):
    i = pl.program_id(0)
    ze = ze_ref[...]
    zq = jnp.concatenate([zqa_ref[...], zqb_ref[...]], axis=1)
    diff = zq - ze
    out_ref[...] = ze + diff
    part = jnp.sum(diff * diff).reshape(1, 1)

    @pl.when(i == 0)
    def _init():
        loss_ref[...] = part

    @pl.when(i != 0)
    def _acc():
        loss_ref[...] = loss_ref[...] + part


def _st_loss(z_qa, z_qb, z_e):
    return pl.pallas_call(
        _st_body,
        grid=(NTOT // _BE,),
        in_specs=[
            pl.BlockSpec((_BE, _DH), lambda i: (i, 0)),
            pl.BlockSpec((_BE, _DH), lambda i: (i, 0)),
            pl.BlockSpec((_BE, D), lambda i: (i, 0)),
        ],
        out_specs=[
            pl.BlockSpec((_BE, D), lambda i: (i, 0)),
            pl.BlockSpec((1, 1), lambda i: (0, 0)),
        ],
        out_shape=[
            jax.ShapeDtypeStruct((NTOT, D), jnp.float32),
            jax.ShapeDtypeStruct((1, 1), jnp.float32),
        ],
    )(z_qa, z_qb, z_e)


def kernel(z_e, codebook_tensor_pca, W, b):
    mc, cn = _map_codebook(codebook_tensor_pca, W, b)
    z_slot = (z_e.reshape(BITEMS, WORD, D).transpose(1, 0, 2)
              .reshape(NTOT, D))
    vals_s, idxs_s = _topk(z_slot, mc, cn.reshape(1, KCB))
    vals = (vals_s.reshape(WORD, BITEMS, PADK).transpose(1, 0, 2)
            .reshape(NTOT, PADK))
    idxs = (idxs_s.reshape(WORD, BITEMS, PADK).transpose(1, 0, 2)
            .reshape(NTOT, PADK))
    sel = _select(vals, idxs)
    idx2 = sel[:, :WORD].reshape(1, NTOT)
    z_qa = _gather_rows(mc[:, :_DH], idx2)
    z_qb = _gather_rows(mc[:, _DH:], idx2)
    z_q_st, loss_sum = _st_loss(z_qa, z_qb, z_e)
    m = loss_sum[0, 0] / jnp.float32(NTOT * D)
    loss = 0.75 * m + 0.25 * m
    return (z_q_st, loss)


# speedup vs baseline: 4.5796x; 1.1648x over previous
import jax
import jax.numpy as jnp
from jax.experimental import pallas as pl
from jax.experimental.pallas import tpu as pltpu
from jax.experimental.pallas import tpu_sc as plsc

WORD = 10
D = 512
P = 4096
KCB = 8192
NTOT = 5120
BITEMS = 512
TOPK = 10
PADK = 16


_RK = 512


def _mc_body(c_ref, w_ref, b_ref, mc_ref, cn_ref):
    mc = jax.lax.dot_general(
        c_ref[...], w_ref[...], (((1,), (1,)), ((), ())),
        preferred_element_type=jnp.float32)
    mc = mc + b_ref[...]
    mc_ref[...] = mc
    cn_ref[...] = jnp.sum(mc * mc, axis=1, keepdims=True)


def _map_codebook(cpca, w, b):
    return pl.pallas_call(
        _mc_body,
        grid=(KCB // _RK,),
        in_specs=[
            pl.BlockSpec((_RK, P), lambda i: (i, 0)),
            pl.BlockSpec((D, P), lambda i: (0, 0)),
            pl.BlockSpec((1, D), lambda i: (0, 0)),
        ],
        out_specs=[
            pl.BlockSpec((_RK, D), lambda i: (i, 0)),
            pl.BlockSpec((_RK, 1), lambda i: (i, 0)),
        ],
        out_shape=[
            jax.ShapeDtypeStruct((KCB, D), jnp.float32),
            jax.ShapeDtypeStruct((KCB, 1), jnp.float32),
        ],
    )(cpca, w, b.reshape(1, D))


_BN = 512
_CH = 1024
_NCH = KCB // _CH


def _make_slot_body(ranks):
    def body(z_ref, mc_ref, cn_ref, vals_ref, idx_ref,
             scr_ref, cvals_ref, cidx_ref):
        k = pl.program_id(0)
        inf = jnp.float32(jnp.inf)
        bigi = jnp.int32(2**30)
        lane = jax.lax.broadcasted_iota(jnp.int32, (_BN, PADK), 1)

        buf = (k + 1) & 1
        chunk = jax.lax.rem(k + 7, _NCH)
        gio = (jax.lax.broadcasted_iota(jnp.int32, (_BN, _CH), 1)
               + chunk * _CH)
        cv = jnp.full((_BN, PADK), inf, jnp.float32)
        ci = jnp.full((_BN, PADK), bigi, jnp.int32)
        for r in range(ranks):
            d = scr_ref[buf]
            cm = jnp.min(d, axis=1, keepdims=True)
            ca = jnp.min(jnp.where(d == cm, gio, bigi), axis=1,
                         keepdims=True)
            cv = jnp.where(lane == r, cm, cv)
            ci = jnp.where(lane == r, ca, ci)
            if r < ranks - 1:
                scr_ref[buf] = jnp.where(gio == ca, inf, d)
        cvals_ref[chunk] = cv
        cidx_ref[chunk] = ci

        z = z_ref[...]
        dot = jax.lax.dot_general(
            z, mc_ref[...], (((1,), (1,)), ((), ())),
            preferred_element_type=jnp.float32)
        zn = jnp.sum(z * z, axis=1, keepdims=True)
        scr_ref[k & 1] = (zn + cn_ref[...]) - 2.0 * dot

        @pl.when(k == _NCH)
        def _merge():
            allv = jnp.concatenate([cvals_ref[c] for c in range(_NCH)],
                                   axis=1)
            alli = jnp.concatenate([cidx_ref[c] for c in range(_NCH)],
                                   axis=1)
            vacc = jnp.full((_BN, PADK), inf, jnp.float32)
            iacc = jnp.zeros((_BN, PADK), jnp.int32)
            for r in range(ranks):
                m = jnp.min(allv, axis=1, keepdims=True)
                w = jnp.min(jnp.where(allv == m, alli, bigi), axis=1,
                            keepdims=True)
                vacc = jnp.where(lane == r, m, vacc)
                iacc = jnp.where(lane == r, w, iacc)
                if r < ranks - 1:
                    allv = jnp.where(alli == w, inf, allv)
            vals_ref[...] = vacc
            idx_ref[...] = iacc

    return body


def _topk(z_slot, mc, cn_row):
    outs_v, outs_i = [], []
    for s in range(WORD):
        v, i = pl.pallas_call(
            _make_slot_body(s + 1),
            grid=(_NCH + 1,),
            in_specs=[
                pl.BlockSpec((_BN, D), lambda k, s=s: (s, 0)),
                pl.BlockSpec((_CH, D), lambda k: (jnp.minimum(k, _NCH - 1), 0)),
                pl.BlockSpec((1, _CH), lambda k: (0, jnp.minimum(k, _NCH - 1))),
            ],
            out_specs=[
                pl.BlockSpec((_BN, PADK), lambda k: (0, 0)),
                pl.BlockSpec((_BN, PADK), lambda k: (0, 0)),
            ],
            out_shape=[
                jax.ShapeDtypeStruct((_BN, PADK), jnp.float32),
                jax.ShapeDtypeStruct((_BN, PADK), jnp.int32),
            ],
            scratch_shapes=[
                pltpu.VMEM((2, _BN, _CH), jnp.float32),
                pltpu.VMEM((_NCH, _BN, PADK), jnp.float32),
                pltpu.VMEM((_NCH, _BN, PADK), jnp.int32),
            ],
        )(z_slot, mc, cn_row)
        outs_v.append(v)
        outs_i.append(i)
    return jnp.concatenate(outs_v, axis=0), jnp.concatenate(outs_i, axis=0)


def _select_body(vals_ref, idx_ref, out_ref):
    inf = jnp.float32(jnp.inf)
    bigi = jnp.int32(2**30)
    lane = jax.lax.broadcasted_iota(jnp.int32, (BITEMS, PADK), 1)
    acc = jnp.zeros((BITEMS, PADK), jnp.int32)
    chosen = []
    for i in range(WORD):
        cv = vals_ref[:, PADK * i:PADK * (i + 1)]
        ci = idx_ref[:, PADK * i:PADK * (i + 1)]
        banned = jnp.zeros(cv.shape, jnp.bool_)
        for cj in chosen:
            banned = banned | (ci == cj)
        mv = jnp.where(banned, inf, cv)
        m = jnp.min(mv, axis=1, keepdims=True)
        pick = jnp.min(jnp.where(mv == m, ci, bigi), axis=1, keepdims=True)
        chosen.append(pick)
        acc = jnp.where(lane == i, pick, acc)
    out_ref[...] = acc


def _select(vals, idxs):
    return pl.pallas_call(
        _select_body,
        grid=(1,),
        in_specs=[
            pl.BlockSpec((BITEMS, WORD * PADK), lambda i: (0, 0)),
            pl.BlockSpec((BITEMS, WORD * PADK), lambda i: (0, 0)),
        ],
        out_specs=pl.BlockSpec((BITEMS, PADK), lambda i: (0, 0)),
        out_shape=jax.ShapeDtypeStruct((BITEMS, PADK), jnp.int32),
    )(vals.reshape(BITEMS, WORD * PADK), idxs.reshape(BITEMS, WORD * PADK))


_GW = 128
_DH = D // 2


def _gather_rows(mc_half, idx2):
    @pl.kernel(
        out_type=jax.ShapeDtypeStruct((NTOT, _DH), jnp.float32),
        mesh=plsc.VectorSubcoreMesh(core_axis_name="core",
                                    subcore_axis_name="subcore"),
    )
    def _k(mc_hbm, i_hbm, o_hbm):
        def body(i_vmem, o_vmem):
            pltpu.sync_copy(mc_hbm.at[i_vmem.at[0]], o_vmem)

        pltpu.emit_pipeline(
            body,
            grid=(NTOT // _GW,),
            in_specs=[pl.BlockSpec((1, _GW), index_map=lambda i: (0, i))],
            out_specs=[pl.BlockSpec((_GW, _DH), index_map=lambda i: (i, 0))],
            core_axis_name="subcore",
            dimension_semantics=(pltpu.PARALLEL,),
        )(i_hbm, o_hbm)

    return _k(mc_half, idx2)


_BE = 512


def _st_body(zqa_ref, zqb_ref, ze_ref, out_ref, loss_ref):
    i = pl.program_id(0)
    ze = ze_ref[...]
    zq = jnp.concatenate([zqa_ref[...], zqb_ref[...]], axis=1)
    diff = zq - ze
    out_ref[...] = ze + diff
    part = jnp.sum(diff * diff).reshape(1, 1)

    @pl.when(i == 0)
    def _init():
        loss_ref[...] = part

    @pl.when(i != 0)
    def _acc():
        loss_ref[...] = loss_ref[...] + part


def _st_loss(z_qa, z_qb, z_e):
    return pl.pallas_call(
        _st_body,
        grid=(NTOT // _BE,),
        in_specs=[
            pl.BlockSpec((_BE, _DH), lambda i: (i, 0)),
            pl.BlockSpec((_BE, _DH), lambda i: (i, 0)),
            pl.BlockSpec((_BE, D), lambda i: (i, 0)),
        ],
        out_specs=[
            pl.BlockSpec((_BE, D), lambda i: (i, 0)),
            pl.BlockSpec((1, 1), lambda i: (0, 0)),
        ],
        out_shape=[
            jax.ShapeDtypeStruct((NTOT, D), jnp.float32),
            jax.ShapeDtypeStruct((1, 1), jnp.float32),
        ],
    )(z_qa, z_qb, z_e)


def kernel(z_e, codebook_tensor_pca, W, b):
    mc, cn = _map_codebook(codebook_tensor_pca, W, b)
    z_slot = (z_e.reshape(BITEMS, WORD, D).transpose(1, 0, 2)
              .reshape(NTOT, D))
    vals_s, idxs_s = _topk(z_slot, mc, cn.reshape(1, KCB))
    vals = (vals_s.reshape(WORD, BITEMS, PADK).transpose(1, 0, 2)
            .reshape(NTOT, PADK))
    idxs = (idxs_s.reshape(WORD, BITEMS, PADK).transpose(1, 0, 2)
            .reshape(NTOT, PADK))
    sel = _select(vals, idxs)
    idx2 = sel[:, :WORD].reshape(1, NTOT)
    z_qa = _gather_rows(mc[:, :_DH], idx2)
    z_qb = _gather_rows(mc[:, _DH:], idx2)
    z_q_st, loss_sum = _st_loss(z_qa, z_qb, z_e)
    m = loss_sum[0, 0] / jnp.float32(NTOT * D)
    loss = 0.75 * m + 0.25 * m
    return (z_q_st, loss)


# speedup vs baseline: 4.8073x; 1.0497x over previous
import jax
import jax.numpy as jnp
from jax.experimental import pallas as pl
from jax.experimental.pallas import tpu as pltpu
from jax.experimental.pallas import tpu_sc as plsc

WORD = 10
D = 512
P = 4096
KCB = 8192
NTOT = 5120
BITEMS = 512
TOPK = 10
PADK = 16


_RK = 512


def _mc_body(c_ref, w_ref, b_ref, mc_ref, cn_ref):
    mc = jax.lax.dot_general(
        c_ref[...], w_ref[...], (((1,), (1,)), ((), ())),
        preferred_element_type=jnp.float32)
    mc = mc + b_ref[...]
    mc_ref[...] = mc
    cn_ref[...] = jnp.sum(mc * mc, axis=1, keepdims=True)


def _map_codebook(cpca, w, b):
    return pl.pallas_call(
        _mc_body,
        grid=(KCB // _RK,),
        in_specs=[
            pl.BlockSpec((_RK, P), lambda i: (i, 0)),
            pl.BlockSpec((D, P), lambda i: (0, 0)),
            pl.BlockSpec((1, D), lambda i: (0, 0)),
        ],
        out_specs=[
            pl.BlockSpec((_RK, D), lambda i: (i, 0)),
            pl.BlockSpec((_RK, 1), lambda i: (i, 0)),
        ],
        out_shape=[
            jax.ShapeDtypeStruct((KCB, D), jnp.float32),
            jax.ShapeDtypeStruct((KCB, 1), jnp.float32),
        ],
    )(cpca, w, b.reshape(1, D))


_BN = 512
_CH = 1024
_NCH = KCB // _CH


def _make_slot_body(ranks):
    def body(z_ref, mc_ref, cn_ref, vals_ref, idx_ref,
             pv_ref, pi_ref, qv_ref, qi_ref, cvals_ref, cidx_ref):
        k = pl.program_id(0)
        inf = jnp.float32(jnp.inf)
        bigi = jnp.int32(2**30)
        lane = jax.lax.broadcasted_iota(jnp.int32, (_BN, PADK), 1)

        buf = (k + 1) & 1
        chunk = jax.lax.rem(k + 7, _NCH)
        pv = pv_ref[buf]
        pi = pi_ref[buf]
        qv = qv_ref[buf]
        qi = qi_ref[buf]
        cv = jnp.full((_BN, PADK), inf, jnp.float32)
        ci = jnp.full((_BN, PADK), bigi, jnp.int32)
        for r in range(ranks):
            cm = jnp.min(pv, axis=1, keepdims=True)
            ca = jnp.min(jnp.where(pv == cm, pi, bigi), axis=1,
                         keepdims=True)
            cv = jnp.where(lane == r, cm, cv)
            ci = jnp.where(lane == r, ca, ci)
            if r < ranks - 1:
                hit = pi == ca
                pv = jnp.where(hit, qv, pv)
                pi = jnp.where(hit, qi, pi)
                qv = jnp.where(hit, inf, qv)
        cvals_ref[chunk] = cv
        cidx_ref[chunk] = ci

        z = z_ref[...]
        dot = jax.lax.dot_general(
            z, mc_ref[...], (((1,), (1,)), ((), ())),
            preferred_element_type=jnp.float32)
        zn = jnp.sum(z * z, axis=1, keepdims=True)
        d = (zn + cn_ref[...]) - 2.0 * dot
        da = d[:, :_CH // 2]
        db = d[:, _CH // 2:]
        base = jnp.minimum(k, _NCH - 1) * _CH
        gioa = (jax.lax.broadcasted_iota(jnp.int32, (_BN, _CH // 2), 1)
                + base)
        giob = gioa + _CH // 2
        c = db < da
        wbuf = k & 1
        pv_ref[wbuf] = jnp.where(c, db, da)
        pi_ref[wbuf] = jnp.where(c, giob, gioa)
        qv_ref[wbuf] = jnp.where(c, da, db)
        qi_ref[wbuf] = jnp.where(c, gioa, giob)

        @pl.when(k == _NCH)
        def _merge():
            allv = jnp.concatenate([cvals_ref[c] for c in range(_NCH)],
                                   axis=1)
            alli = jnp.concatenate([cidx_ref[c] for c in range(_NCH)],
                                   axis=1)
            vacc = jnp.full((_BN, PADK), inf, jnp.float32)
            iacc = jnp.zeros((_BN, PADK), jnp.int32)
            for r in range(ranks):
                m = jnp.min(allv, axis=1, keepdims=True)
                w = jnp.min(jnp.where(allv == m, alli, bigi), axis=1,
                            keepdims=True)
                vacc = jnp.where(lane == r, m, vacc)
                iacc = jnp.where(lane == r, w, iacc)
                if r < ranks - 1:
                    allv = jnp.where(alli == w, inf, allv)
            vals_ref[...] = vacc
            idx_ref[...] = iacc

    return body


def _topk(z_slot, mc, cn_row):
    outs_v, outs_i = [], []
    for s in range(WORD):
        v, i = pl.pallas_call(
            _make_slot_body(s + 1),
            grid=(_NCH + 1,),
            in_specs=[
                pl.BlockSpec((_BN, D), lambda k, s=s: (s, 0)),
                pl.BlockSpec((_CH, D), lambda k: (jnp.minimum(k, _NCH - 1), 0)),
                pl.BlockSpec((1, _CH), lambda k: (0, jnp.minimum(k, _NCH - 1))),
            ],
            out_specs=[
                pl.BlockSpec((_BN, PADK), lambda k: (0, 0)),
                pl.BlockSpec((_BN, PADK), lambda k: (0, 0)),
            ],
            out_shape=[
                jax.ShapeDtypeStruct((_BN, PADK), jnp.float32),
                jax.ShapeDtypeStruct((_BN, PADK), jnp.int32),
            ],
            scratch_shapes=[
                pltpu.VMEM((2, _BN, _CH // 2), jnp.float32),
                pltpu.VMEM((2, _BN, _CH // 2), jnp.int32),
                pltpu.VMEM((2, _BN, _CH // 2), jnp.float32),
                pltpu.VMEM((2, _BN, _CH // 2), jnp.int32),
                pltpu.VMEM((_NCH, _BN, PADK), jnp.float32),
                pltpu.VMEM((_NCH, _BN, PADK), jnp.int32),
            ],
        )(z_slot, mc, cn_row)
        outs_v.append(v)
        outs_i.append(i)
    return jnp.concatenate(outs_v, axis=0), jnp.concatenate(outs_i, axis=0)


def _select_body(vals_ref, idx_ref, out_ref):
    inf = jnp.float32(jnp.inf)
    bigi = jnp.int32(2**30)
    lane = jax.lax.broadcasted_iota(jnp.int32, (BITEMS, PADK), 1)
    acc = jnp.zeros((BITEMS, PADK), jnp.int32)
    chosen = []
    for i in range(WORD):
        cv = vals_ref[:, PADK * i:PADK * (i + 1)]
        ci = idx_ref[:, PADK * i:PADK * (i + 1)]
        banned = jnp.zeros(cv.shape, jnp.bool_)
        for cj in chosen:
            banned = banned | (ci == cj)
        mv = jnp.where(banned, inf, cv)
        m = jnp.min(mv, axis=1, keepdims=True)
        pick = jnp.min(jnp.where(mv == m, ci, bigi), axis=1, keepdims=True)
        chosen.append(pick)
        acc = jnp.where(lane == i, pick, acc)
    out_ref[...] = acc


def _select(vals, idxs):
    return pl.pallas_call(
        _select_body,
        grid=(1,),
        in_specs=[
            pl.BlockSpec((BITEMS, WORD * PADK), lambda i: (0, 0)),
            pl.BlockSpec((BITEMS, WORD * PADK), lambda i: (0, 0)),
        ],
        out_specs=pl.BlockSpec((BITEMS, PADK), lambda i: (0, 0)),
        out_shape=jax.ShapeDtypeStruct((BITEMS, PADK), jnp.int32),
    )(vals.reshape(BITEMS, WORD * PADK), idxs.reshape(BITEMS, WORD * PADK))


_GW = 128
_DH = D // 2


def _gather_rows(mc_half, idx2):
    @pl.kernel(
        out_type=jax.ShapeDtypeStruct((NTOT, _DH), jnp.float32),
        mesh=plsc.VectorSubcoreMesh(core_axis_name="core",
                                    subcore_axis_name="subcore"),
    )
    def _k(mc_hbm, i_hbm, o_hbm):
        def body(i_vmem, o_vmem):
            pltpu.sync_copy(mc_hbm.at[i_vmem.at[0]], o_vmem)

        pltpu.emit_pipeline(
            body,
            grid=(NTOT // _GW,),
            in_specs=[pl.BlockSpec((1, _GW), index_map=lambda i: (0, i))],
            out_specs=[pl.BlockSpec((_GW, _DH), index_map=lambda i: (i, 0))],
            core_axis_name="subcore",
            dimension_semantics=(pltpu.PARALLEL,),
        )(i_hbm, o_hbm)

    return _k(mc_half, idx2)


_BE = 512


def _st_body(zqa_ref, zqb_ref, ze_ref, out_ref, loss_ref):
    i = pl.program_id(0)
    ze = ze_ref[...]
    zq = jnp.concatenate([zqa_ref[...], zqb_ref[...]], axis=1)
    diff = zq - ze
    out_ref[...] = ze + diff
    part = jnp.sum(diff * diff).reshape(1, 1)

    @pl.when(i == 0)
    def _init():
        loss_ref[...] = part

    @pl.when(i != 0)
    def _acc():
        loss_ref[...] = loss_ref[...] + part


def _st_loss(z_qa, z_qb, z_e):
    return pl.pallas_call(
        _st_body,
        grid=(NTOT // _BE,),
        in_specs=[
            pl.BlockSpec((_BE, _DH), lambda i: (i, 0)),
            pl.BlockSpec((_BE, _DH), lambda i: (i, 0)),
            pl.BlockSpec((_BE, D), lambda i: (i, 0)),
        ],
        out_specs=[
            pl.BlockSpec((_BE, D), lambda i: (i, 0)),
            pl.BlockSpec((1, 1), lambda i: (0, 0)),
        ],
        out_shape=[
            jax.ShapeDtypeStruct((NTOT, D), jnp.float32),
            jax.ShapeDtypeStruct((1, 1), jnp.float32),
        ],
    )(z_qa, z_qb, z_e)


def kernel(z_e, codebook_tensor_pca, W, b):
    mc, cn = _map_codebook(codebook_tensor_pca, W, b)
    z_slot = (z_e.reshape(BITEMS, WORD, D).transpose(1, 0, 2)
              .reshape(NTOT, D))
    vals_s, idxs_s = _topk(z_slot, mc, cn.reshape(1, KCB))
    vals = (vals_s.reshape(WORD, BITEMS, PADK).transpose(1, 0, 2)
            .reshape(NTOT, PADK))
    idxs = (idxs_s.reshape(WORD, BITEMS, PADK).transpose(1, 0, 2)
            .reshape(NTOT, PADK))
    sel = _select(vals, idxs)
    idx2 = sel[:, :WORD].reshape(1, NTOT)
    z_qa = _gather_rows(mc[:, :_DH], idx2)
    z_qb = _gather_rows(mc[:, _DH:], idx2)
    z_q_st, loss_sum = _st_loss(z_qa, z_qb, z_e)
    m = loss_sum[0, 0] / jnp.float32(NTOT * D)
    loss = 0.75 * m + 0.25 * m
    return (z_q_st, loss)
